# baseline (speedup 1.0000x reference)
import jax
import jax.numpy as jnp
from jax.experimental import pallas as pl
from jax.experimental.pallas import tpu as pltpu

ALPHA = 0.25
NEG_LN2 = -0.6931471805599453
NS, NL = 8, 2500


def _focal_block(cls_ref, reg_ref, anc_ref, ann_ref, ign_ref, ann_lab_ref,
                 ann_box_ref, out_ref, xt_ref):
    C = xt_ref.shape[0]
    M = ann_ref.shape[1]
    G = ign_ref.shape[1]

    for s in range(NS):
        xt_ref[:, s, :] = jnp.transpose(
            cls_ref[0, s * NL:(s + 1) * NL, :], (1, 0))

    ax0 = anc_ref[0, 0]
    ay0 = anc_ref[0, 1]
    ax1 = anc_ref[0, 2]
    ay1 = anc_ref[0, 3]
    aw = ax1 - ax0
    ah = ay1 - ay0
    area_a = aw * ah

    ann = ann_ref[0]
    bx0 = ann[:, 0].reshape(M, 1, 1)
    by0 = ann[:, 1].reshape(M, 1, 1)
    bx1 = ann[:, 2].reshape(M, 1, 1)
    by1 = ann[:, 3].reshape(M, 1, 1)
    iw = jnp.maximum(jnp.minimum(ax1, bx1) - jnp.maximum(ax0, bx0), 0.0)
    ih = jnp.maximum(jnp.minimum(ay1, by1) - jnp.maximum(ay0, by0), 0.0)
    inter = iw * ih
    ua = jnp.maximum(area_a + (bx1 - bx0) * (by1 - by0) - inter, 1e-8)
    iou = inter / ua
    iou_max = jnp.max(iou, axis=0, keepdims=True)
    m_iota = jax.lax.broadcasted_iota(jnp.int32, iou.shape, 0)
    argmax = jnp.min(jnp.where(iou == iou_max, m_iota, M), axis=0)
    iou_max = iou_max[0]

    ign = ign_ref[0]
    gx0 = ign[:, 0].reshape(G, 1, 1)
    gy0 = ign[:, 1].reshape(G, 1, 1)
    gx1 = ign[:, 2].reshape(G, 1, 1)
    gy1 = ign[:, 3].reshape(G, 1, 1)
    giw = jnp.maximum(jnp.minimum(ax1, gx1) - jnp.maximum(ax0, gx0), 0.0)
    gih = jnp.maximum(jnp.minimum(ay1, gy1) - jnp.maximum(ay0, gy0), 0.0)
    ginter = giw * gih
    gua = jnp.maximum(area_a + (gx1 - gx0) * (gy1 - gy0) - ginter, 1e-8)
    keep = jnp.max(ginter / gua, axis=0) < 0.5

    pos = (iou_max >= 0.5) & keep
    neg = (iou_max < 0.4) & keep
    posf = pos.astype(jnp.float32)
    num_pos = jnp.sum(posf)

    x_l = xt_ref[ann_lab_ref[0, 0, 0]]
    gx0a = jnp.full((NS, NL), ann_box_ref[0, 0, 0], jnp.float32)
    gy0a = jnp.full((NS, NL), ann_box_ref[0, 0, 1], jnp.float32)
    gx1a = jnp.full((NS, NL), ann_box_ref[0, 0, 2], jnp.float32)
    gy1a = jnp.full((NS, NL), ann_box_ref[0, 0, 3], jnp.float32)
    for m in range(1, M):
        is_m = argmax == m
        x_l = jnp.where(is_m, xt_ref[ann_lab_ref[0, 0, m]], x_l)
        gx0a = jnp.where(is_m, ann_box_ref[0, m, 0], gx0a)
        gy0a = jnp.where(is_m, ann_box_ref[0, m, 1], gy0a)
        gx1a = jnp.where(is_m, ann_box_ref[0, m, 2], gx1a)
        gy1a = jnp.where(is_m, ann_box_ref[0, m, 3], gy1a)

    x = jnp.minimum(xt_ref[...], 1.0 - 1e-4)
    t0 = (x * x) * jnp.log2(1.0 - x)
    col_sum = jnp.sum(t0, axis=0)

    x_l = jnp.clip(x_l, 1e-4, 1.0 - 1e-4)
    base = col_sum * ((1.0 - ALPHA) * NEG_LN2)
    t0_l = (1.0 - ALPHA) * x_l * x_l * (-jnp.log(1.0 - x_l))
    t1_l = ALPHA * (1.0 - x_l) * (1.0 - x_l) * (-jnp.log(x_l))
    row_loss = jnp.where(pos, base - t0_l + t1_l,
                         jnp.where(neg, base, 0.0))
    cls_sum = jnp.sum(row_loss)

    gw_raw = gx1a - gx0a
    gh_raw = gy1a - gy0a
    gcx = gx0a + 0.5 * gw_raw
    gcy = gy0a + 0.5 * gh_raw
    gw = jnp.maximum(gw_raw, 1.0)
    gh = jnp.maximum(gh_raw, 1.0)
    acx = ax0 + 0.5 * aw
    acy = ay0 + 0.5 * ah
    t_0 = ((gcx - acx) / aw) / 0.1
    t_1 = ((gcy - acy) / ah) / 0.1
    t_2 = jnp.log(gw / aw) / 0.2
    t_3 = jnp.log(gh / ah) / 0.2

    def smooth_l1(t, r):
        d = jnp.abs(t - r)
        return jnp.where(d <= 1.0 / 9.0, 0.5 * 9.0 * d * d, d - 0.5 / 9.0)

    rl = (smooth_l1(t_0, reg_ref[0, 0]) +
          smooth_l1(t_1, reg_ref[0, 1]) +
          smooth_l1(t_2, reg_ref[0, 2]) +
          smooth_l1(t_3, reg_ref[0, 3]))
    reg_sum = jnp.sum(rl * posf)

    s_iota = jax.lax.broadcasted_iota(jnp.int32, (8, 128), 0)
    l_iota = jax.lax.broadcasted_iota(jnp.int32, (8, 128), 1)
    lane0 = l_iota == 0
    out_ref[0] = (jnp.where(lane0 & (s_iota == 0), cls_sum, 0.0) +
                  jnp.where(lane0 & (s_iota == 1), reg_sum, 0.0) +
                  jnp.where(lane0 & (s_iota == 2), num_pos, 0.0))


@jax.jit
def kernel(classifications, regressions, anchors, annotations, ignores):
    B, N, C = classifications.shape
    M = annotations.shape[1]
    G = ignores.shape[1]
    cls_v = classifications
    reg_t = jnp.transpose(regressions.reshape(B, NS, NL, 4), (0, 3, 1, 2))
    anc_t = jnp.transpose(anchors.reshape(1, NS, NL, 4), (0, 3, 1, 2))
    ann_lab = annotations[:, :, 4].astype(jnp.int32).reshape(B, 1, M)
    ann_box = annotations[:, :, :4]

    out = pl.pallas_call(
        _focal_block,
        grid=(B,),
        in_specs=[
            pl.BlockSpec((1, N, C), lambda j: (j, 0, 0)),
            pl.BlockSpec((1, 4, NS, NL), lambda j: (j, 0, 0, 0)),
            pl.BlockSpec((1, 4, NS, NL), lambda j: (0, 0, 0, 0)),
            pl.BlockSpec((1, M, 5), lambda j: (j, 0, 0)),
            pl.BlockSpec((1, G, 5), lambda j: (j, 0, 0)),
            pl.BlockSpec((1, 1, M), lambda j: (j, 0, 0),
                         memory_space=pltpu.SMEM),
            pl.BlockSpec((1, M, 4), lambda j: (j, 0, 0),
                         memory_space=pltpu.SMEM),
        ],
        out_specs=pl.BlockSpec((1, 8, 128), lambda j: (j, 0, 0)),
        out_shape=jax.ShapeDtypeStruct((B, 8, 128), jnp.float32),
        scratch_shapes=[pltpu.VMEM((C, NS, NL), jnp.float32)],
        compiler_params=pltpu.CompilerParams(
            dimension_semantics=("arbitrary",)),
    )(cls_v, reg_t, anc_t, annotations, ignores, ann_lab, ann_box)

    cls_sums = out[:, 0, 0]
    reg_sums = out[:, 1, 0]
    npos = out[:, 2, 0]
    cls_losses = cls_sums / jnp.maximum(npos, 1.0)
    reg_losses = reg_sums / jnp.maximum(npos * 4.0, 1.0)
    return jnp.stack([jnp.mean(cls_losses), jnp.mean(reg_losses)])


# speedup vs baseline: 2.4522x; 2.4522x over previous
import jax
import jax.numpy as jnp
from jax.experimental import pallas as pl
from jax.experimental.pallas import tpu as pltpu

ALPHA = 0.25
NEG_LN2 = -0.6931471805599453
NS, NL = 8, 2500


def _focal_block(cls_ref, reg_ref, anc_ref, ann_ref, ign_ref, ann_lab_ref,
                 ann_box_ref, out_ref):
    C = cls_ref.shape[1]
    M = ann_ref.shape[1]
    G = ign_ref.shape[1]

    ax0 = anc_ref[0, 0]
    ay0 = anc_ref[0, 1]
    ax1 = anc_ref[0, 2]
    ay1 = anc_ref[0, 3]
    aw = ax1 - ax0
    ah = ay1 - ay0
    area_a = aw * ah

    ann = ann_ref[0]
    bx0 = ann[:, 0].reshape(M, 1, 1)
    by0 = ann[:, 1].reshape(M, 1, 1)
    bx1 = ann[:, 2].reshape(M, 1, 1)
    by1 = ann[:, 3].reshape(M, 1, 1)
    iw = jnp.maximum(jnp.minimum(ax1, bx1) - jnp.maximum(ax0, bx0), 0.0)
    ih = jnp.maximum(jnp.minimum(ay1, by1) - jnp.maximum(ay0, by0), 0.0)
    inter = iw * ih
    ua = jnp.maximum(area_a + (bx1 - bx0) * (by1 - by0) - inter, 1e-8)
    iou = inter / ua
    iou_max = jnp.max(iou, axis=0, keepdims=True)
    m_iota = jax.lax.broadcasted_iota(jnp.int32, iou.shape, 0)
    argmax = jnp.min(jnp.where(iou == iou_max, m_iota, M), axis=0)
    iou_max = iou_max[0]

    ign = ign_ref[0]
    gx0 = ign[:, 0].reshape(G, 1, 1)
    gy0 = ign[:, 1].reshape(G, 1, 1)
    gx1 = ign[:, 2].reshape(G, 1, 1)
    gy1 = ign[:, 3].reshape(G, 1, 1)
    giw = jnp.maximum(jnp.minimum(ax1, gx1) - jnp.maximum(ax0, gx0), 0.0)
    gih = jnp.maximum(jnp.minimum(ay1, gy1) - jnp.maximum(ay0, gy0), 0.0)
    ginter = giw * gih
    gua = jnp.maximum(area_a + (gx1 - gx0) * (gy1 - gy0) - ginter, 1e-8)
    keep = jnp.max(ginter / gua, axis=0) < 0.5

    pos = (iou_max >= 0.5) & keep
    neg = (iou_max < 0.4) & keep
    posf = pos.astype(jnp.float32)
    num_pos = jnp.sum(posf)

    lab8 = jnp.full((NS, NL), ann_lab_ref[0, 0, 0], jnp.int32)
    gx0a = jnp.full((NS, NL), ann_box_ref[0, 0, 0], jnp.float32)
    gy0a = jnp.full((NS, NL), ann_box_ref[0, 0, 1], jnp.float32)
    gx1a = jnp.full((NS, NL), ann_box_ref[0, 0, 2], jnp.float32)
    gy1a = jnp.full((NS, NL), ann_box_ref[0, 0, 3], jnp.float32)
    for m in range(1, M):
        is_m = argmax == m
        lab8 = jnp.where(is_m, ann_lab_ref[0, 0, m], lab8)
        gx0a = jnp.where(is_m, ann_box_ref[0, m, 0], gx0a)
        gy0a = jnp.where(is_m, ann_box_ref[0, m, 1], gy0a)
        gx1a = jnp.where(is_m, ann_box_ref[0, m, 2], gx1a)
        gy1a = jnp.where(is_m, ann_box_ref[0, m, 3], gy1a)

    x = jnp.minimum(cls_ref[0], 1.0 - 1e-4)
    t0 = (x * x) * jnp.log2(1.0 - x)
    col_sum2 = jnp.sum(t0, axis=0, keepdims=True)
    labels2 = jnp.concatenate([lab8[s:s + 1, :] for s in range(NS)], axis=1)
    c_iota = jax.lax.broadcasted_iota(jnp.int32, x.shape, 0)
    x_l2 = jnp.sum(jnp.where(c_iota == labels2, x, 0.0), axis=0,
                   keepdims=True)
    col_sum = jnp.concatenate(
        [col_sum2[:, s * NL:(s + 1) * NL] for s in range(NS)], axis=0)
    x_l = jnp.concatenate(
        [x_l2[:, s * NL:(s + 1) * NL] for s in range(NS)], axis=0)

    x_l = jnp.clip(x_l, 1e-4, 1.0 - 1e-4)
    base = col_sum * ((1.0 - ALPHA) * NEG_LN2)
    t0_l = (1.0 - ALPHA) * x_l * x_l * (-jnp.log(1.0 - x_l))
    t1_l = ALPHA * (1.0 - x_l) * (1.0 - x_l) * (-jnp.log(x_l))
    row_loss = jnp.where(pos, base - t0_l + t1_l,
                         jnp.where(neg, base, 0.0))
    cls_sum = jnp.sum(row_loss)

    gw_raw = gx1a - gx0a
    gh_raw = gy1a - gy0a
    gcx = gx0a + 0.5 * gw_raw
    gcy = gy0a + 0.5 * gh_raw
    gw = jnp.maximum(gw_raw, 1.0)
    gh = jnp.maximum(gh_raw, 1.0)
    acx = ax0 + 0.5 * aw
    acy = ay0 + 0.5 * ah
    t_0 = ((gcx - acx) / aw) / 0.1
    t_1 = ((gcy - acy) / ah) / 0.1
    t_2 = jnp.log(gw / aw) / 0.2
    t_3 = jnp.log(gh / ah) / 0.2

    def smooth_l1(t, r):
        d = jnp.abs(t - r)
        return jnp.where(d <= 1.0 / 9.0, 0.5 * 9.0 * d * d, d - 0.5 / 9.0)

    rl = (smooth_l1(t_0, reg_ref[0, 0]) +
          smooth_l1(t_1, reg_ref[0, 1]) +
          smooth_l1(t_2, reg_ref[0, 2]) +
          smooth_l1(t_3, reg_ref[0, 3]))
    reg_sum = jnp.sum(rl * posf)

    s_iota = jax.lax.broadcasted_iota(jnp.int32, (8, 128), 0)
    l_iota = jax.lax.broadcasted_iota(jnp.int32, (8, 128), 1)
    lane0 = l_iota == 0
    out_ref[0] = (jnp.where(lane0 & (s_iota == 0), cls_sum, 0.0) +
                  jnp.where(lane0 & (s_iota == 1), reg_sum, 0.0) +
                  jnp.where(lane0 & (s_iota == 2), num_pos, 0.0))


@jax.jit
def kernel(classifications, regressions, anchors, annotations, ignores):
    B, N, C = classifications.shape
    M = annotations.shape[1]
    G = ignores.shape[1]
    cls_t = jnp.transpose(classifications, (0, 2, 1))
    reg_t = jnp.transpose(regressions, (0, 2, 1)).reshape(B, 4, NS, NL)
    anc_t = jnp.transpose(anchors, (0, 2, 1)).reshape(1, 4, NS, NL)
    ann_lab = annotations[:, :, 4].astype(jnp.int32).reshape(B, 1, M)
    ann_box = annotations[:, :, :4]

    out = pl.pallas_call(
        _focal_block,
        grid=(B,),
        in_specs=[
            pl.BlockSpec((1, C, N), lambda j: (j, 0, 0)),
            pl.BlockSpec((1, 4, NS, NL), lambda j: (j, 0, 0, 0)),
            pl.BlockSpec((1, 4, NS, NL), lambda j: (0, 0, 0, 0)),
            pl.BlockSpec((1, M, 5), lambda j: (j, 0, 0)),
            pl.BlockSpec((1, G, 5), lambda j: (j, 0, 0)),
            pl.BlockSpec((1, 1, M), lambda j: (j, 0, 0),
                         memory_space=pltpu.SMEM),
            pl.BlockSpec((1, M, 4), lambda j: (j, 0, 0),
                         memory_space=pltpu.SMEM),
        ],
        out_specs=pl.BlockSpec((1, 8, 128), lambda j: (j, 0, 0)),
        out_shape=jax.ShapeDtypeStruct((B, 8, 128), jnp.float32),
        compiler_params=pltpu.CompilerParams(
            dimension_semantics=("arbitrary",)),
    )(cls_t, reg_t, anc_t, annotations, ignores, ann_lab, ann_box)

    cls_sums = out[:, 0, 0]
    reg_sums = out[:, 1, 0]
    npos = out[:, 2, 0]
    cls_losses = cls_sums / jnp.maximum(npos, 1.0)
    reg_losses = reg_sums / jnp.maximum(npos * 4.0, 1.0)
    return jnp.stack([jnp.mean(cls_losses), jnp.mean(reg_losses)])


# speedup vs baseline: 2.6286x; 1.0719x over previous
import jax
import jax.numpy as jnp
from jax.experimental import pallas as pl
from jax.experimental.pallas import tpu as pltpu

ALPHA = 0.25
NEG_LN2 = -0.6931471805599453
NS, NL = 8, 2500


def _focal_block(cls_ref, reg_ref, anc_ref, ann_ref, ign_ref, ann_lab_ref,
                 ann_box_ref, out_ref):
    C = cls_ref.shape[1]
    M = ann_ref.shape[1]
    G = ign_ref.shape[1]

    ax0 = anc_ref[0, 0]
    ay0 = anc_ref[0, 1]
    ax1 = anc_ref[0, 2]
    ay1 = anc_ref[0, 3]
    aw = ax1 - ax0
    ah = ay1 - ay0
    area_a = aw * ah

    ann = ann_ref[0]
    bx0 = ann[:, 0].reshape(M, 1, 1)
    by0 = ann[:, 1].reshape(M, 1, 1)
    bx1 = ann[:, 2].reshape(M, 1, 1)
    by1 = ann[:, 3].reshape(M, 1, 1)
    iw = jnp.maximum(jnp.minimum(ax1, bx1) - jnp.maximum(ax0, bx0), 0.0)
    ih = jnp.maximum(jnp.minimum(ay1, by1) - jnp.maximum(ay0, by0), 0.0)
    inter = iw * ih
    ua = jnp.maximum(area_a + (bx1 - bx0) * (by1 - by0) - inter, 1e-8)
    iou = inter / ua
    iou_max = jnp.max(iou, axis=0, keepdims=True)
    m_iota = jax.lax.broadcasted_iota(jnp.int32, iou.shape, 0)
    argmax = jnp.min(jnp.where(iou == iou_max, m_iota, M), axis=0)
    iou_max = iou_max[0]

    ign = ign_ref[0]
    gx0 = ign[:, 0].reshape(G, 1, 1)
    gy0 = ign[:, 1].reshape(G, 1, 1)
    gx1 = ign[:, 2].reshape(G, 1, 1)
    gy1 = ign[:, 3].reshape(G, 1, 1)
    giw = jnp.maximum(jnp.minimum(ax1, gx1) - jnp.maximum(ax0, gx0), 0.0)
    gih = jnp.maximum(jnp.minimum(ay1, gy1) - jnp.maximum(ay0, gy0), 0.0)
    ginter = giw * gih
    gua = jnp.maximum(area_a + (gx1 - gx0) * (gy1 - gy0) - ginter, 1e-8)
    keep = jnp.max(ginter / gua, axis=0) < 0.5

    pos = (iou_max >= 0.5) & keep
    neg = (iou_max < 0.4) & keep
    posf = pos.astype(jnp.float32)
    num_pos = jnp.sum(posf)

    lab8 = jnp.full((NS, NL), ann_lab_ref[0, 0, 0], jnp.int32)
    gx0a = jnp.full((NS, NL), ann_box_ref[0, 0, 0], jnp.float32)
    gy0a = jnp.full((NS, NL), ann_box_ref[0, 0, 1], jnp.float32)
    gx1a = jnp.full((NS, NL), ann_box_ref[0, 0, 2], jnp.float32)
    gy1a = jnp.full((NS, NL), ann_box_ref[0, 0, 3], jnp.float32)
    for m in range(1, M):
        is_m = argmax == m
        lab8 = jnp.where(is_m, ann_lab_ref[0, 0, m], lab8)
        gx0a = jnp.where(is_m, ann_box_ref[0, m, 0], gx0a)
        gy0a = jnp.where(is_m, ann_box_ref[0, m, 1], gy0a)
        gx1a = jnp.where(is_m, ann_box_ref[0, m, 2], gx1a)
        gy1a = jnp.where(is_m, ann_box_ref[0, m, 3], gy1a)

    x = jnp.minimum(cls_ref[0], 1.0 - 1e-4)
    t0 = (x * x) * jnp.log2(1.0 - x)
    ones_c = jnp.ones((1, C), jnp.float32)
    col_sum2 = jax.lax.dot_general(
        ones_c, t0, (((1,), (0,)), ((), ())),
        preferred_element_type=jnp.float32)
    labels2 = jnp.concatenate([lab8[s:s + 1, :] for s in range(NS)], axis=1)
    c_iota = jax.lax.broadcasted_iota(jnp.int32, x.shape, 0)
    x_l2 = jax.lax.dot_general(
        ones_c, jnp.where(c_iota == labels2, x, 0.0), (((1,), (0,)), ((), ())),
        preferred_element_type=jnp.float32)
    col_sum = jnp.concatenate(
        [col_sum2[:, s * NL:(s + 1) * NL] for s in range(NS)], axis=0)
    x_l = jnp.concatenate(
        [x_l2[:, s * NL:(s + 1) * NL] for s in range(NS)], axis=0)

    x_l = jnp.clip(x_l, 1e-4, 1.0 - 1e-4)
    base = col_sum * ((1.0 - ALPHA) * NEG_LN2)
    t0_l = (1.0 - ALPHA) * x_l * x_l * (-jnp.log(1.0 - x_l))
    t1_l = ALPHA * (1.0 - x_l) * (1.0 - x_l) * (-jnp.log(x_l))
    row_loss = jnp.where(pos, base - t0_l + t1_l,
                         jnp.where(neg, base, 0.0))
    cls_sum = jnp.sum(row_loss)

    gw_raw = gx1a - gx0a
    gh_raw = gy1a - gy0a
    gcx = gx0a + 0.5 * gw_raw
    gcy = gy0a + 0.5 * gh_raw
    gw = jnp.maximum(gw_raw, 1.0)
    gh = jnp.maximum(gh_raw, 1.0)
    acx = ax0 + 0.5 * aw
    acy = ay0 + 0.5 * ah
    t_0 = ((gcx - acx) / aw) / 0.1
    t_1 = ((gcy - acy) / ah) / 0.1
    t_2 = jnp.log(gw / aw) / 0.2
    t_3 = jnp.log(gh / ah) / 0.2

    def smooth_l1(t, r):
        d = jnp.abs(t - r)
        return jnp.where(d <= 1.0 / 9.0, 0.5 * 9.0 * d * d, d - 0.5 / 9.0)

    rl = (smooth_l1(t_0, reg_ref[0, 0]) +
          smooth_l1(t_1, reg_ref[0, 1]) +
          smooth_l1(t_2, reg_ref[0, 2]) +
          smooth_l1(t_3, reg_ref[0, 3]))
    reg_sum = jnp.sum(rl * posf)

    s_iota = jax.lax.broadcasted_iota(jnp.int32, (8, 128), 0)
    l_iota = jax.lax.broadcasted_iota(jnp.int32, (8, 128), 1)
    lane0 = l_iota == 0
    out_ref[0] = (jnp.where(lane0 & (s_iota == 0), cls_sum, 0.0) +
                  jnp.where(lane0 & (s_iota == 1), reg_sum, 0.0) +
                  jnp.where(lane0 & (s_iota == 2), num_pos, 0.0))


@jax.jit
def kernel(classifications, regressions, anchors, annotations, ignores):
    B, N, C = classifications.shape
    M = annotations.shape[1]
    G = ignores.shape[1]
    cls_t = jnp.transpose(classifications, (0, 2, 1))
    reg_t = jnp.transpose(regressions, (0, 2, 1)).reshape(B, 4, NS, NL)
    anc_t = jnp.transpose(anchors, (0, 2, 1)).reshape(1, 4, NS, NL)
    ann_lab = annotations[:, :, 4].astype(jnp.int32).reshape(B, 1, M)
    ann_box = annotations[:, :, :4]

    out = pl.pallas_call(
        _focal_block,
        grid=(B,),
        in_specs=[
            pl.BlockSpec((1, C, N), lambda j: (j, 0, 0)),
            pl.BlockSpec((1, 4, NS, NL), lambda j: (j, 0, 0, 0)),
            pl.BlockSpec((1, 4, NS, NL), lambda j: (0, 0, 0, 0)),
            pl.BlockSpec((1, M, 5), lambda j: (j, 0, 0)),
            pl.BlockSpec((1, G, 5), lambda j: (j, 0, 0)),
            pl.BlockSpec((1, 1, M), lambda j: (j, 0, 0),
                         memory_space=pltpu.SMEM),
            pl.BlockSpec((1, M, 4), lambda j: (j, 0, 0),
                         memory_space=pltpu.SMEM),
        ],
        out_specs=pl.BlockSpec((1, 8, 128), lambda j: (j, 0, 0)),
        out_shape=jax.ShapeDtypeStruct((B, 8, 128), jnp.float32),
        compiler_params=pltpu.CompilerParams(
            dimension_semantics=("arbitrary",)),
    )(cls_t, reg_t, anc_t, annotations, ignores, ann_lab, ann_box)

    cls_sums = out[:, 0, 0]
    reg_sums = out[:, 1, 0]
    npos = out[:, 2, 0]
    cls_losses = cls_sums / jnp.maximum(npos, 1.0)
    reg_losses = reg_sums / jnp.maximum(npos * 4.0, 1.0)
    return jnp.stack([jnp.mean(cls_losses), jnp.mean(reg_losses)])


# speedup vs baseline: 2.7682x; 1.0531x over previous
import jax
import jax.numpy as jnp
from jax.experimental import pallas as pl
from jax.experimental.pallas import tpu as pltpu

ALPHA = 0.25
NEG_LN2 = -0.6931471805599453
NS, NL = 8, 2500


def _focal_block(cls_ref, reg_ref, anc_ref, ann_ref, ign_ref, ann_lab_ref,
                 ann_box_ref, out_ref):
    j = pl.program_id(0)
    B = pl.num_programs(0)
    C = cls_ref.shape[1]
    M = ann_ref.shape[2]
    G = ign_ref.shape[2]

    ax0 = anc_ref[0, 0]
    ay0 = anc_ref[0, 1]
    ax1 = anc_ref[0, 2]
    ay1 = anc_ref[0, 3]
    aw = ax1 - ax0
    ah = ay1 - ay0
    area_a = aw * ah

    annv = ann_ref[...]
    b_iota = jax.lax.broadcasted_iota(jnp.int32, annv.shape, 1)
    ann = jnp.transpose(
        jnp.sum(jnp.where(b_iota == j, annv, 0.0), axis=1), (1, 0))
    bx0 = ann[:, 0].reshape(M, 1, 1)
    by0 = ann[:, 1].reshape(M, 1, 1)
    bx1 = ann[:, 2].reshape(M, 1, 1)
    by1 = ann[:, 3].reshape(M, 1, 1)
    iw = jnp.maximum(jnp.minimum(ax1, bx1) - jnp.maximum(ax0, bx0), 0.0)
    ih = jnp.maximum(jnp.minimum(ay1, by1) - jnp.maximum(ay0, by0), 0.0)
    inter = iw * ih
    ua = jnp.maximum(area_a + (bx1 - bx0) * (by1 - by0) - inter, 1e-8)
    iou = inter / ua
    iou_max = jnp.max(iou, axis=0, keepdims=True)
    m_iota = jax.lax.broadcasted_iota(jnp.int32, iou.shape, 0)
    argmax = jnp.min(jnp.where(iou == iou_max, m_iota, M), axis=0)
    iou_max = iou_max[0]

    ignv = ign_ref[...]
    gb_iota = jax.lax.broadcasted_iota(jnp.int32, ignv.shape, 1)
    ign = jnp.transpose(
        jnp.sum(jnp.where(gb_iota == j, ignv, 0.0), axis=1), (1, 0))
    gx0 = ign[:, 0].reshape(G, 1, 1)
    gy0 = ign[:, 1].reshape(G, 1, 1)
    gx1 = ign[:, 2].reshape(G, 1, 1)
    gy1 = ign[:, 3].reshape(G, 1, 1)
    giw = jnp.maximum(jnp.minimum(ax1, gx1) - jnp.maximum(ax0, gx0), 0.0)
    gih = jnp.maximum(jnp.minimum(ay1, gy1) - jnp.maximum(ay0, gy0), 0.0)
    ginter = giw * gih
    gua = jnp.maximum(area_a + (gx1 - gx0) * (gy1 - gy0) - ginter, 1e-8)
    keep = jnp.max(ginter / gua, axis=0) < 0.5

    pos = (iou_max >= 0.5) & keep
    neg = (iou_max < 0.4) & keep
    posf = pos.astype(jnp.float32)
    num_pos = jnp.sum(posf)

    lab8 = jnp.full((NS, NL), ann_lab_ref[0, 0, 0], jnp.int32)
    gx0a = jnp.full((NS, NL), ann_box_ref[0, 0, 0], jnp.float32)
    gy0a = jnp.full((NS, NL), ann_box_ref[0, 0, 1], jnp.float32)
    gx1a = jnp.full((NS, NL), ann_box_ref[0, 0, 2], jnp.float32)
    gy1a = jnp.full((NS, NL), ann_box_ref[0, 0, 3], jnp.float32)
    for m in range(1, M):
        is_m = argmax == m
        lab8 = jnp.where(is_m, ann_lab_ref[0, 0, m], lab8)
        gx0a = jnp.where(is_m, ann_box_ref[0, m, 0], gx0a)
        gy0a = jnp.where(is_m, ann_box_ref[0, m, 1], gy0a)
        gx1a = jnp.where(is_m, ann_box_ref[0, m, 2], gx1a)
        gy1a = jnp.where(is_m, ann_box_ref[0, m, 3], gy1a)

    x = jnp.minimum(cls_ref[0], 1.0 - 1e-4)
    t0 = (x * x) * jnp.log2(1.0 - x)
    ones_c = jnp.ones((1, C), jnp.float32)
    col_sum2 = jax.lax.dot_general(
        ones_c, t0, (((1,), (0,)), ((), ())),
        preferred_element_type=jnp.float32)
    labels2 = jnp.concatenate([lab8[s:s + 1, :] for s in range(NS)], axis=1)
    c_iota = jax.lax.broadcasted_iota(jnp.int32, x.shape, 0)
    x_l2 = jax.lax.dot_general(
        ones_c, jnp.where(c_iota == labels2, x, 0.0), (((1,), (0,)), ((), ())),
        preferred_element_type=jnp.float32)
    col_sum = jnp.concatenate(
        [col_sum2[:, s * NL:(s + 1) * NL] for s in range(NS)], axis=0)
    x_l = jnp.concatenate(
        [x_l2[:, s * NL:(s + 1) * NL] for s in range(NS)], axis=0)

    x_l = jnp.clip(x_l, 1e-4, 1.0 - 1e-4)
    base = col_sum * ((1.0 - ALPHA) * NEG_LN2)
    t0_l = (1.0 - ALPHA) * x_l * x_l * (-jnp.log(1.0 - x_l))
    t1_l = ALPHA * (1.0 - x_l) * (1.0 - x_l) * (-jnp.log(x_l))
    row_loss = jnp.where(pos, base - t0_l + t1_l,
                         jnp.where(neg, base, 0.0))
    cls_sum = jnp.sum(row_loss)

    gw_raw = gx1a - gx0a
    gh_raw = gy1a - gy0a
    gcx = gx0a + 0.5 * gw_raw
    gcy = gy0a + 0.5 * gh_raw
    gw = jnp.maximum(gw_raw, 1.0)
    gh = jnp.maximum(gh_raw, 1.0)
    acx = ax0 + 0.5 * aw
    acy = ay0 + 0.5 * ah
    t_0 = ((gcx - acx) / aw) / 0.1
    t_1 = ((gcy - acy) / ah) / 0.1
    t_2 = jnp.log(gw / aw) / 0.2
    t_3 = jnp.log(gh / ah) / 0.2

    def smooth_l1(t, r):
        d = jnp.abs(t - r)
        return jnp.where(d <= 1.0 / 9.0, 0.5 * 9.0 * d * d, d - 0.5 / 9.0)

    rl = (smooth_l1(t_0, reg_ref[0, 0]) +
          smooth_l1(t_1, reg_ref[0, 1]) +
          smooth_l1(t_2, reg_ref[0, 2]) +
          smooth_l1(t_3, reg_ref[0, 3]))
    reg_sum = jnp.sum(rl * posf)

    cls_loss_j = cls_sum / jnp.maximum(num_pos, 1.0)
    reg_loss_j = reg_sum / jnp.maximum(num_pos * 4.0, 1.0)
    s_iota = jax.lax.broadcasted_iota(jnp.int32, (2, 128), 0)
    l_iota = jax.lax.broadcasted_iota(jnp.int32, (2, 128), 1)
    lane0 = l_iota == 0
    inv_b = 1.0 / B
    vec = (jnp.where(lane0 & (s_iota == 0), cls_loss_j * inv_b, 0.0) +
           jnp.where(lane0 & (s_iota == 1), reg_loss_j * inv_b, 0.0))

    @pl.when(j == 0)
    def _():
        out_ref[0] = jnp.zeros((2, 128), jnp.float32)

    out_ref[0] += vec


@jax.jit
def kernel(classifications, regressions, anchors, annotations, ignores):
    B, N, C = classifications.shape
    M = annotations.shape[1]
    G = ignores.shape[1]
    cls_t = jnp.transpose(classifications, (0, 2, 1))
    reg_t = jnp.transpose(regressions, (0, 2, 1)).reshape(B, 4, NS, NL)
    anc_t = jnp.transpose(anchors, (0, 2, 1)).reshape(1, 4, NS, NL)
    ann_t = jnp.transpose(annotations, (2, 0, 1))
    ign_t = jnp.transpose(ignores, (2, 0, 1))
    ann_lab = annotations[:, :, 4].astype(jnp.int32).reshape(B, 1, M)
    ann_box = annotations[:, :, :4]

    out = pl.pallas_call(
        _focal_block,
        grid=(B,),
        in_specs=[
            pl.BlockSpec((1, C, N), lambda j: (j, 0, 0)),
            pl.BlockSpec((1, 4, NS, NL), lambda j: (j, 0, 0, 0)),
            pl.BlockSpec((1, 4, NS, NL), lambda j: (0, 0, 0, 0)),
            pl.BlockSpec((5, B, M), lambda j: (0, 0, 0)),
            pl.BlockSpec((5, B, G), lambda j: (0, 0, 0)),
            pl.BlockSpec((1, 1, M), lambda j: (j, 0, 0),
                         memory_space=pltpu.SMEM),
            pl.BlockSpec((1, M, 4), lambda j: (j, 0, 0),
                         memory_space=pltpu.SMEM),
        ],
        out_specs=pl.BlockSpec((1, 2, 128), lambda j: (0, 0, 0)),
        out_shape=jax.ShapeDtypeStruct((1, 2, 128), jnp.float32),
        compiler_params=pltpu.CompilerParams(
            dimension_semantics=("arbitrary",)),
    )(cls_t, reg_t, anc_t, ann_t, ign_t, ann_lab, ann_box)

    return out[0, :, 0]
